# baseline (speedup 1.0000x reference)
import jax
import jax.numpy as jnp
from jax.experimental import pallas as pl

N_NODES = 10000
D = 256


def _mm_body(a_ref, w_ref, b_ref, o_ref):
    acc = jnp.dot(a_ref[...], w_ref[...], preferred_element_type=jnp.float32)
    o_ref[...] = jnp.maximum(acc + b_ref[...], 0.0)


def _tc_linear_relu(agg, W, b):
    blk = 1000
    return pl.pallas_call(
        _mm_body,
        grid=(N_NODES // blk,),
        in_specs=[
            pl.BlockSpec((blk, D), lambda i: (i, 0)),
            pl.BlockSpec((D, D), lambda i: (0, 0)),
            pl.BlockSpec((1, D), lambda i: (0, 0)),
        ],
        out_specs=pl.BlockSpec((blk, D), lambda i: (i, 0)),
        out_shape=jax.ShapeDtypeStruct((N_NODES, D), jnp.float32),
    )(agg, W, b.reshape(1, D))


def kernel(x, edge_index, W, b):
    src = edge_index[0]
    dst = edge_index[1]
    msg = jnp.take(x, src, axis=0)
    agg = jax.ops.segment_sum(msg, dst, num_segments=N_NODES)
    return _tc_linear_relu(agg, W, b)
